# Initial kernel scaffold; baseline (speedup 1.0000x reference)
import functools

import jax
import jax.numpy as jnp
from jax import lax
from jax.experimental import pallas as pl
from jax.experimental.pallas import tpu as pltpu
from jax.experimental.pallas import tpu_sc as plsc

E = 8
TOPK = 2
D = 1024
F = 2048
N = 4096
T = 256
NB = N // T
G = NB + E - 1

NW = 32
CHUNK = N // NW
GROWS = 64
LANES = 16


def _route_body(keys_hbm, hs_hbm, perm_hbm, cnt_hbm,
                keys_v, sidx_v, src_v, rows_v, cnt_v, sem):
    wid = lax.axis_index("s") * 2 + lax.axis_index("c")
    pbase = wid * CHUNK
    lanes = lax.iota(jnp.int32, LANES)

    pltpu.sync_copy(keys_hbm, keys_v)

    def hist_body(v, accs):
        kv = keys_v[pl.ds(v * LANES, LANES)]
        return tuple(accs[e] + jnp.where(kv == e, 1, 0).astype(jnp.int32)
                     for e in range(E))

    accs = lax.fori_loop(0, N // LANES, hist_body,
                         tuple(jnp.zeros((LANES,), jnp.int32) for _ in range(E)))
    totals = [jnp.sum(a) for a in accs]
    offs = [jnp.int32(0)]
    for e in range(E):
        offs.append(offs[-1] + totals[e])

    tv = jnp.zeros((LANES,), jnp.int32)
    for e in range(E):
        tv = jnp.where(lanes == e, totals[e], tv)
    cnt_v[...] = tv

    @pl.when(wid == 0)
    def _():
        pltpu.sync_copy(cnt_v, cnt_hbm)

    for e in range(E):
        lo = offs[e]
        hi = offs[e + 1]

        @pl.when((hi > pbase) & (lo < pbase + CHUNK))
        def _(e=e, lo=lo):
            def pos_body(v, rc):
                kv = keys_v[pl.ds(v * LANES, LANES)]
                m = kv == e
                mi = jnp.where(m, 1, 0).astype(jnp.int32)
                cs = plsc.cumsum(mi)
                p = lo + rc + cs - 1
                inr = m & (p >= pbase) & (p < pbase + CHUNK)
                plsc.store_scatter(sidx_v, [p - pbase], lanes + v * LANES,
                                   mask=inr)
                return rc + jnp.sum(mi)

            lax.fori_loop(0, N // LANES, pos_body, jnp.int32(0))

    for c in range(CHUNK // GROWS):
        for v in range(GROWS // LANES):
            sv = sidx_v[pl.ds(c * GROWS + v * LANES, LANES)]
            src_v[pl.ds(v * LANES, LANES)] = lax.shift_right_logical(sv, 1)
        pltpu.async_copy(hs_hbm.at[src_v], rows_v, sem).wait()
        pltpu.sync_copy(rows_v, perm_hbm.at[pl.ds(pbase + c * GROWS, GROWS)])


@functools.partial(
    pl.kernel,
    out_type=(
        jax.ShapeDtypeStruct((N, D), jnp.float32),
        jax.ShapeDtypeStruct((LANES,), jnp.int32),
    ),
    mesh=plsc.VectorSubcoreMesh(core_axis_name="c", subcore_axis_name="s"),
    scratch_types=[
        pltpu.VMEM((N,), jnp.int32),
        pltpu.VMEM((CHUNK,), jnp.int32),
        pltpu.VMEM((GROWS,), jnp.int32),
        pltpu.VMEM((GROWS, D), jnp.float32),
        pltpu.VMEM((LANES,), jnp.int32),
        pltpu.SemaphoreType.DMA,
    ],
)
def _route(keys_hbm, hs_hbm, perm_hbm, cnt_hbm, *scratch):
    _route_body(keys_hbm, hs_hbm, perm_hbm, cnt_hbm, *scratch)


def _gmm_body(gids, tids, offs, x_ref, w1_ref, w2_ref, out_ref):
    s = pl.program_id(0)
    e = gids[s]
    t = tids[s]
    row0 = t * T
    lo = jnp.clip(offs[e] - row0, 0, T)
    hi = jnp.clip(offs[e + 1] - row0, 0, T)

    x = x_ref[...]
    h = jnp.dot(x, w1_ref[0], preferred_element_type=jnp.float32)
    a = h[:, :F]
    b = h[:, F:]
    inter = (a * jax.nn.sigmoid(a)) * b
    y = jnp.dot(inter, w2_ref[0], preferred_element_type=jnp.float32)

    rows = lax.broadcasted_iota(jnp.int32, (T, 1), 0)
    m = (rows >= lo) & (rows < hi)
    is_first = jnp.logical_or(s == 0, tids[jnp.maximum(s - 1, 0)] != t)

    @pl.when(is_first)
    def _():
        out_ref[...] = jnp.where(m, y, 0.0)

    @pl.when(jnp.logical_not(is_first))
    def _():
        out_ref[...] = jnp.where(m, y, out_ref[...])


def kernel(hidden_states, tokens_per_expert, w1, w2):
    hs = hidden_states.reshape(-1, D)
    keys = tokens_per_expert.reshape(-1)

    permuted, cnt16 = _route(keys, hs)
    counts = cnt16[:E]

    offsets = jnp.concatenate(
        [jnp.zeros((1,), jnp.int32), jnp.cumsum(counts, dtype=jnp.int32)])
    t_start = offsets[:E] // T
    t_end = jnp.where(counts > 0, (offsets[1:] + T - 1) // T, t_start)
    num_t = t_end - t_start
    cum = jnp.cumsum(num_t, dtype=jnp.int32)
    total = cum[E - 1]
    s_eff = jnp.minimum(jnp.arange(G, dtype=jnp.int32), total - 1)
    gids = jnp.searchsorted(cum, s_eff, side="right").astype(jnp.int32)
    tids = (t_start[gids] + (s_eff - (cum[gids] - num_t[gids]))).astype(jnp.int32)

    grid_spec = pltpu.PrefetchScalarGridSpec(
        num_scalar_prefetch=3,
        grid=(G,),
        in_specs=[
            pl.BlockSpec((T, D), lambda s, gids, tids, offs: (tids[s], 0)),
            pl.BlockSpec((1, D, 2 * F), lambda s, gids, tids, offs: (gids[s], 0, 0)),
            pl.BlockSpec((1, F, D), lambda s, gids, tids, offs: (gids[s], 0, 0)),
        ],
        out_specs=pl.BlockSpec((T, D), lambda s, gids, tids, offs: (tids[s], 0)),
    )
    out = pl.pallas_call(
        _gmm_body,
        grid_spec=grid_spec,
        out_shape=jax.ShapeDtypeStruct((N, D), jnp.float32),
    )(gids, tids, offsets, permuted, w1, w2)
    return out


# speedup vs baseline: 4.0436x; 4.0436x over previous
import functools

import jax
import jax.numpy as jnp
from jax import lax
from jax.experimental import pallas as pl
from jax.experimental.pallas import tpu as pltpu
from jax.experimental.pallas import tpu_sc as plsc

E = 8
TOPK = 2
D = 1024
F = 2048
N = 4096
T = 256
NB = N // T
G = NB + E - 1

NW = 32
CHUNK = N // NW
GROWS = 64
LANES = 16


def _route_body(keys_hbm, hs_hbm, perm_hbm, cnt_hbm,
                keys_v, sidx_v, src_v, rows_v, cnt_v, sem):
    wid = lax.axis_index("s") * 2 + lax.axis_index("c")
    pbase = wid * CHUNK
    lanes = lax.iota(jnp.int32, LANES)

    pltpu.sync_copy(keys_hbm, keys_v)

    def hist_body(v, accs):
        kv = keys_v[pl.ds(v * LANES, LANES)]
        return tuple(accs[e] + jnp.where(kv == e, 1, 0).astype(jnp.int32)
                     for e in range(E))

    accs = lax.fori_loop(0, N // LANES, hist_body,
                         tuple(jnp.zeros((LANES,), jnp.int32) for _ in range(E)))
    totals = [jnp.sum(a) for a in accs]
    offs = [jnp.int32(0)]
    for e in range(E):
        offs.append(offs[-1] + totals[e])

    tv = jnp.zeros((LANES,), jnp.int32)
    for e in range(E):
        tv = jnp.where(lanes == e, totals[e], tv)
    cnt_v[...] = tv

    @pl.when(wid == 0)
    def _():
        pltpu.sync_copy(cnt_v, cnt_hbm)

    for e in range(E):
        lo = offs[e]
        hi = offs[e + 1]

        @pl.when((hi > pbase) & (lo < pbase + CHUNK))
        def _(e=e, lo=lo):
            def pos_body(v, rc):
                kv = keys_v[pl.ds(v * LANES, LANES)]
                m = kv == e
                mi = jnp.where(m, 1, 0).astype(jnp.int32)
                cs = plsc.cumsum(mi)
                p = lo + rc + cs - 1
                inr = m & (p >= pbase) & (p < pbase + CHUNK)
                plsc.store_scatter(sidx_v, [p - pbase], lanes + v * LANES,
                                   mask=inr)
                return rc + jnp.sum(mi)

            lax.fori_loop(0, N // LANES, pos_body, jnp.int32(0))

    for c in range(CHUNK // GROWS):
        for v in range(GROWS // LANES):
            sv = sidx_v[pl.ds(c * GROWS + v * LANES, LANES)]
            src_v[pl.ds(v * LANES, LANES)] = lax.shift_right_logical(sv, 1)
        pltpu.async_copy(hs_hbm.at[src_v], rows_v, sem).wait()
        pltpu.sync_copy(rows_v, perm_hbm.at[pl.ds(pbase + c * GROWS, GROWS)])


@functools.cache
def _make_route():
    return pl.kernel(
        _route_body,
        out_type=(
            jax.ShapeDtypeStruct((N, D), jnp.float32),
            jax.ShapeDtypeStruct((LANES,), jnp.int32),
        ),
        mesh=plsc.VectorSubcoreMesh(core_axis_name="c", subcore_axis_name="s"),
        scratch_types=[
            pltpu.VMEM((N,), jnp.int32),
            pltpu.VMEM((CHUNK,), jnp.int32),
            pltpu.VMEM((GROWS,), jnp.int32),
            pltpu.VMEM((GROWS, D), jnp.float32),
            pltpu.VMEM((LANES,), jnp.int32),
            pltpu.SemaphoreType.DMA,
        ],
        compiler_params=pltpu.CompilerParams(needs_layout_passes=False),
    )


def _gmm_body(gids, tids, offs, x_ref, w1_ref, w2_ref, out_ref):
    s = pl.program_id(0)
    e = gids[s]
    t = tids[s]
    row0 = t * T
    lo = jnp.clip(offs[e] - row0, 0, T)
    hi = jnp.clip(offs[e + 1] - row0, 0, T)

    x = x_ref[...]
    h = jnp.dot(x, w1_ref[0], preferred_element_type=jnp.float32)
    a = h[:, :F]
    b = h[:, F:]
    inter = (a * jax.nn.sigmoid(a)) * b
    y = jnp.dot(inter, w2_ref[0], preferred_element_type=jnp.float32)

    rows = lax.broadcasted_iota(jnp.int32, (T, 1), 0)
    m = (rows >= lo) & (rows < hi)
    is_first = jnp.logical_or(s == 0, tids[jnp.maximum(s - 1, 0)] != t)

    @pl.when(is_first)
    def _():
        out_ref[...] = jnp.where(m, y, 0.0)

    @pl.when(jnp.logical_not(is_first))
    def _():
        out_ref[...] = jnp.where(m, y, out_ref[...])


def kernel(hidden_states, tokens_per_expert, w1, w2):
    hs = hidden_states.reshape(-1, D)
    keys = tokens_per_expert.reshape(-1)

    permuted, cnt16 = _make_route()(keys, hs)
    counts = cnt16[:E]

    offsets = jnp.concatenate(
        [jnp.zeros((1,), jnp.int32), jnp.cumsum(counts, dtype=jnp.int32)])
    t_start = offsets[:E] // T
    t_end = jnp.where(counts > 0, (offsets[1:] + T - 1) // T, t_start)
    num_t = t_end - t_start
    cum = jnp.cumsum(num_t, dtype=jnp.int32)
    total = cum[E - 1]
    s_eff = jnp.minimum(jnp.arange(G, dtype=jnp.int32), total - 1)
    gids = jnp.searchsorted(cum, s_eff, side="right").astype(jnp.int32)
    tids = (t_start[gids] + (s_eff - (cum[gids] - num_t[gids]))).astype(jnp.int32)

    grid_spec = pltpu.PrefetchScalarGridSpec(
        num_scalar_prefetch=3,
        grid=(G,),
        in_specs=[
            pl.BlockSpec((T, D), lambda s, gids, tids, offs: (tids[s], 0)),
            pl.BlockSpec((1, D, 2 * F), lambda s, gids, tids, offs: (gids[s], 0, 0)),
            pl.BlockSpec((1, F, D), lambda s, gids, tids, offs: (gids[s], 0, 0)),
        ],
        out_specs=pl.BlockSpec((T, D), lambda s, gids, tids, offs: (tids[s], 0)),
    )
    out = pl.pallas_call(
        _gmm_body,
        grid_spec=grid_spec,
        out_shape=jax.ShapeDtypeStruct((N, D), jnp.float32),
    )(gids, tids, offsets, permuted, w1, w2)
    return out
